# baseline (speedup 1.0000x reference)
import jax
import jax.numpy as jnp
from jax.experimental import pallas as pl
from jax.experimental.pallas import tpu as pltpu

_B = 16384
_K = 8192
_DL = 32
_BB = 256


def _fused_kernel(x_ref, w1_ref, b1_ref, w2_ref, b2_ref,
                  dw1_ref, db1_ref, dw2_ref, db2_ref, emb_ref,
                  xr_ref, loss_ref):
    i = pl.program_id(0)
    x = x_ref[...]
    h = jnp.maximum(
        jnp.dot(x, w1_ref[...], preferred_element_type=jnp.float32)
        + b1_ref[...], 0.0)
    z = (jnp.dot(h, w2_ref[...], preferred_element_type=jnp.float32)
         + b2_ref[...])

    emb = emb_ref[...]
    e2 = jnp.sum(emb * emb, axis=1)
    scores = (-2.0) * jax.lax.dot_general(
        z, emb, (((1,), (1,)), ((), ())),
        preferred_element_type=jnp.float32) + e2[None, :]
    idx = jnp.argmin(scores, axis=1)

    onehot = (jax.lax.broadcasted_iota(jnp.int32, (_BB, _K), 1)
              == idx[:, None]).astype(jnp.float32)
    z_q = jnp.dot(onehot, emb, preferred_element_type=jnp.float32)

    diff = z_q - z
    part = jnp.sum(diff * diff).reshape(1, 1)

    @pl.when(i == 0)
    def _init():
        loss_ref[...] = jnp.zeros((1, 1), jnp.float32)

    loss_ref[...] += part

    hd = jnp.maximum(
        jnp.dot(z_q, dw1_ref[...], preferred_element_type=jnp.float32)
        + db1_ref[...], 0.0)
    xr_ref[...] = jax.nn.sigmoid(
        jnp.dot(hd, dw2_ref[...], preferred_element_type=jnp.float32)
        + db2_ref[...])


def kernel(x, enc_w1, enc_b1, enc_w2, enc_b2,
           dec_w1, dec_b1, dec_w2, dec_b2, emb):
    b, d_in = x.shape
    d_h = enc_w1.shape[1]
    d_l = enc_w2.shape[1]
    k = emb.shape[0]
    grid = (b // _BB,)

    full = lambda shape: pl.BlockSpec(shape, lambda i: (0, 0))
    x_recon, loss = pl.pallas_call(
        _fused_kernel,
        grid=grid,
        in_specs=[
            pl.BlockSpec((_BB, d_in), lambda i: (i, 0)),
            full((d_in, d_h)),
            full((1, d_h)),
            full((d_h, d_l)),
            full((1, d_l)),
            full((d_l, d_h)),
            full((1, d_h)),
            full((d_h, d_in)),
            full((1, d_in)),
            full((k, d_l)),
        ],
        out_specs=[
            pl.BlockSpec((_BB, d_in), lambda i: (i, 0)),
            pl.BlockSpec((1, 1), lambda i: (0, 0)),
        ],
        out_shape=[
            jax.ShapeDtypeStruct((b, d_in), jnp.float32),
            jax.ShapeDtypeStruct((1, 1), jnp.float32),
        ],
    )(x, enc_w1, enc_b1.reshape(1, -1), enc_w2, enc_b2.reshape(1, -1),
      dec_w1, dec_b1.reshape(1, -1), dec_w2, dec_b2.reshape(1, -1), emb)

    vq_loss = loss[0, 0] * (1.25 / (b * d_l))
    return (x_recon, vq_loss)


# speedup vs baseline: 1.3025x; 1.3025x over previous
import jax
import jax.numpy as jnp
from jax.experimental import pallas as pl
from jax.experimental.pallas import tpu as pltpu

_BB = 256


def _fused_kernel(x_ref, w1_ref, b1_ref, w2_ref, b2_ref,
                  dw1_ref, db1_ref, dw2_ref, db2_ref, emb_ref,
                  xr_ref, loss_ref):
    i = pl.program_id(0)
    x = x_ref[...]
    h = jnp.maximum(
        jnp.dot(x, w1_ref[...], preferred_element_type=jnp.float32)
        + b1_ref[...], 0.0)
    z = (jnp.dot(h, w2_ref[...], preferred_element_type=jnp.float32)
         + b2_ref[...])

    emb = emb_ref[...]
    embh = emb.astype(jnp.bfloat16)
    e2 = jnp.sum(emb * emb, axis=1, keepdims=True)
    z_aug = jnp.concatenate(
        [(-2.0 * z).astype(jnp.bfloat16),
         jnp.ones((_BB, 1), jnp.bfloat16)], axis=1)
    emb_aug = jnp.concatenate([embh, e2.astype(jnp.bfloat16)], axis=1)
    scores = jax.lax.dot_general(
        z_aug, emb_aug, (((1,), (1,)), ((), ())),
        preferred_element_type=jnp.float32)
    m = jnp.min(scores, axis=1, keepdims=True)
    mask = (scores == m).astype(jnp.bfloat16)
    emb_ones = jnp.concatenate(
        [embh, jnp.ones((emb.shape[0], 1), jnp.bfloat16)], axis=1)
    zq_cnt = jnp.dot(mask, emb_ones, preferred_element_type=jnp.float32)
    z_q = zq_cnt[:, :-1] / zq_cnt[:, -1:]

    diff = z_q - z
    part = jnp.sum(diff * diff).reshape(1, 1)

    @pl.when(i == 0)
    def _init():
        loss_ref[...] = jnp.zeros((1, 1), jnp.float32)

    loss_ref[...] += part

    hd = jnp.maximum(
        jnp.dot(z_q, dw1_ref[...], preferred_element_type=jnp.float32)
        + db1_ref[...], 0.0)
    xr_ref[...] = jax.nn.sigmoid(
        jnp.dot(hd, dw2_ref[...], preferred_element_type=jnp.float32)
        + db2_ref[...])


def kernel(x, enc_w1, enc_b1, enc_w2, enc_b2,
           dec_w1, dec_b1, dec_w2, dec_b2, emb):
    b, d_in = x.shape
    d_h = enc_w1.shape[1]
    d_l = enc_w2.shape[1]
    k = emb.shape[0]
    grid = (b // _BB,)

    full = lambda shape: pl.BlockSpec(shape, lambda i: (0, 0))
    x_recon, loss = pl.pallas_call(
        _fused_kernel,
        grid=grid,
        in_specs=[
            pl.BlockSpec((_BB, d_in), lambda i: (i, 0)),
            full((d_in, d_h)),
            full((1, d_h)),
            full((d_h, d_l)),
            full((1, d_l)),
            full((d_l, d_h)),
            full((1, d_h)),
            full((d_h, d_in)),
            full((1, d_in)),
            full((k, d_l)),
        ],
        out_specs=[
            pl.BlockSpec((_BB, d_in), lambda i: (i, 0)),
            pl.BlockSpec((1, 1), lambda i: (0, 0)),
        ],
        out_shape=[
            jax.ShapeDtypeStruct((b, d_in), jnp.float32),
            jax.ShapeDtypeStruct((1, 1), jnp.float32),
        ],
    )(x, enc_w1, enc_b1.reshape(1, -1), enc_w2, enc_b2.reshape(1, -1),
      dec_w1, dec_b1.reshape(1, -1), dec_w2, dec_b2.reshape(1, -1), emb)

    vq_loss = loss[0, 0] * (1.25 / (b * d_l))
    return (x_recon, vq_loss)
